# Initial kernel scaffold; baseline (speedup 1.0000x reference)
import jax
import jax.numpy as jnp
from jax.experimental import pallas as pl


def kernel(emb, q, c, a, qlen, clen, alen):
    raise NotImplementedError("write your pallas kernel here")


# speedup vs baseline: 2.5582x; 2.5582x over previous
import functools

import jax
import jax.numpy as jnp
from jax import lax
from jax.experimental import pallas as pl
from jax.experimental.pallas import tpu as pltpu
from jax.experimental.pallas import tpu_sc as plsc

_NC, _NS, _L = 2, 16, 16
_NW = _NC * _NS

_N = 4096
_B = 32
_LC = 32
_LA = 24
_D = 256
_NITEMS = _N + _B
_NPAD = 4352
_PER_W = _NPAD // _NW
_CH = 8
_NCHUNK = _PER_W // _CH


def _pool_sc_kernel(emb_h, idx_h, coef_h, a_h, out_h, aout_h,
                    idx_v, rows_v, coef_v, acc_v, aidx_v, arows_v, sem):
    w = lax.axis_index("s") * _NC + lax.axis_index("c")
    base = w * _PER_W

    def chunk_body(ci, carry):
        b = base + ci * _CH
        pltpu.sync_copy(idx_h.at[pl.ds(b * _LC, _CH * _LC)], idx_v)
        pltpu.sync_copy(coef_h.at[pl.ds(b, _CH)], coef_v)
        pltpu.async_copy(emb_h.at[idx_v], rows_v, sem).wait()

        def item_body(j, carry2):
            coefj = coef_v[j]
            r0 = j * _LC
            for ch in range(_D // _L):
                sl = pl.ds(ch * _L, _L)
                acc = rows_v[r0, sl] * (1.0 - coefj)
                for l in range(1, _LC):
                    acc = acc + rows_v[r0 + l, sl]
                acc_v[j, sl] = acc
            return carry2

        lax.fori_loop(0, _CH, item_body, 0)
        pltpu.sync_copy(acc_v, out_h.at[pl.ds(b, _CH)])
        return carry

    lax.fori_loop(0, _NCHUNK, chunk_body, 0)

    pltpu.sync_copy(a_h.at[pl.ds(w * _LA, _LA)], aidx_v)
    pltpu.async_copy(emb_h.at[aidx_v], arows_v, sem).wait()
    pltpu.sync_copy(arows_v, aout_h.at[w])


def _gather_top_sc_kernel(emb_h, ctok_h, best_h, cout_h,
                          bidx_v, ctoksel_v, cemb_v, sem):
    w = lax.axis_index("s") * _NC + lax.axis_index("c")
    pltpu.sync_copy(best_h, bidx_v)
    pltpu.async_copy(ctok_h.at[bidx_v], ctoksel_v, sem).wait()
    pltpu.async_copy(emb_h.at[ctoksel_v.at[w, pl.ds(0, _LC)]], cemb_v,
                     sem).wait()
    pltpu.sync_copy(cemb_v, cout_h.at[w])


def _score_tc_kernel(s_ref, best_ref):
    S = s_ref[...]
    cs = S[:_N, :]
    qs = S[_N:_N + _B, :]
    cn = cs * lax.rsqrt(jnp.sum(cs * cs, axis=1, keepdims=True) + 1e-30)
    scores = lax.dot_general(qs, cn, (((1,), (1,)), ((), ())),
                             preferred_element_type=jnp.float32)
    m = jnp.max(scores, axis=1, keepdims=True)
    ii = lax.broadcasted_iota(jnp.int32, scores.shape, 1)
    cand = jnp.where(scores >= m, ii, jnp.int32(2 ** 30))
    best_ref[...] = jnp.min(cand, axis=1)


def _loss_tc_kernel(alen_ref, a_ref, c_ref, out_ref):
    b = pl.program_id(0)
    A = a_ref[0]
    C = c_ref[0]
    an = A * lax.rsqrt(jnp.sum(A * A, axis=1, keepdims=True))
    cn = C * lax.rsqrt(jnp.sum(C * C, axis=1, keepdims=True))
    cos = lax.dot_general(an, cn, (((1,), (1,)), ((), ())),
                          preferred_element_type=jnp.float32)
    em = jnp.exp(-0.5 * (cos - 1.0) ** 2 / (0.001 ** 2))
    sm = em / (jnp.sum(em, axis=1, keepdims=True) + 1e-10)
    mm = jnp.sum(em * sm, axis=1, keepdims=True)
    al = alen_ref[b].astype(jnp.float32)
    mask = (lax.broadcasted_iota(jnp.int32, (_LA, 1), 0)
            < alen_ref[b]).astype(jnp.float32)
    tot = jnp.sum(mm * mask)
    loss_b = 1.0 - tot / al

    @pl.when(b == 0)
    def _():
        out_ref[0, 0] = 0.0

    out_ref[0, 0] += loss_b / _B


def _sc_mesh():
    return plsc.VectorSubcoreMesh(core_axis_name="c", subcore_axis_name="s",
                                  num_cores=_NC, num_subcores=_NS)


def _pool_call(*args):
    return pl.kernel(
        _pool_sc_kernel,
        out_type=(jax.ShapeDtypeStruct((_NPAD, _D), jnp.float32),
                  jax.ShapeDtypeStruct((_B, _LA, _D), jnp.float32)),
        mesh=_sc_mesh(),
        scratch_types=[
        pltpu.VMEM((_CH * _LC,), jnp.int32),
        pltpu.VMEM((_CH * _LC, _D), jnp.float32),
        pltpu.VMEM((_CH, _L), jnp.float32),
        pltpu.VMEM((_CH, _D), jnp.float32),
            pltpu.VMEM((_LA,), jnp.int32),
            pltpu.VMEM((_LA, _D), jnp.float32),
            pltpu.SemaphoreType.DMA,
        ],
    )(*args)


def _gather_top_call(*args):
    return pl.kernel(
        _gather_top_sc_kernel,
        out_type=jax.ShapeDtypeStruct((_B, _LC, _D), jnp.float32),
        mesh=_sc_mesh(),
        scratch_types=[
            pltpu.VMEM((_B,), jnp.int32),
            pltpu.VMEM((_B, 128), jnp.int32),
            pltpu.VMEM((_LC, _D), jnp.float32),
            pltpu.SemaphoreType.DMA,
        ],
    )(*args)


def kernel(emb, q, c, a, qlen, clen, alen):
    emb = emb.astype(jnp.float32)
    ctok = c[:, :, 0].astype(jnp.int32)
    qtok = q[:, :, 0].astype(jnp.int32)
    clen = clen.astype(jnp.int32)
    qlen = qlen.astype(jnp.int32)
    alen = alen.astype(jnp.int32)
    a32 = a.astype(jnp.int32)

    pos = jnp.arange(_LC, dtype=jnp.int32)[None, :]
    cidx = jnp.where(pos < clen[:, None], ctok, ctok[:, :1])
    qidx = jnp.where(pos < qlen[:, None], qtok, qtok[:, :1])
    ccoef = (_LC - clen).astype(jnp.float32)
    qcoef = (_LC - qlen).astype(jnp.float32)

    idx_all = jnp.concatenate(
        [cidx, qidx, jnp.zeros((_NPAD - _NITEMS, _LC), jnp.int32)], axis=0)
    coef_all = jnp.concatenate(
        [ccoef, qcoef, jnp.zeros((_NPAD - _NITEMS,), jnp.float32)], axis=0)
    idx_flat = idx_all.reshape(-1)
    coef_b = coef_all[:, None] + jnp.zeros((_NPAD, _L), jnp.float32)
    a_flat = a32.reshape(-1)

    ssum, a_emb = _pool_call(emb, idx_flat, coef_b, a_flat)

    best = pl.pallas_call(
        _score_tc_kernel,
        out_shape=jax.ShapeDtypeStruct((_B,), jnp.int32),
    )(ssum)

    ctok_pad = jnp.pad(ctok, ((0, 0), (0, 128 - _LC)))
    c_emb = _gather_top_call(emb, ctok_pad, best)

    loss = pl.pallas_call(
        _loss_tc_kernel,
        grid=(_B,),
        in_specs=[
            pl.BlockSpec(memory_space=pltpu.SMEM),
            pl.BlockSpec((1, _LA, _D), lambda b: (b, 0, 0)),
            pl.BlockSpec((1, _LC, _D), lambda b: (b, 0, 0)),
        ],
        out_specs=pl.BlockSpec(memory_space=pltpu.SMEM),
        out_shape=jax.ShapeDtypeStruct((1, 1), jnp.float32),
    )(alen, a_emb, c_emb)

    return loss[0, 0]


# speedup vs baseline: 2.6842x; 1.0493x over previous
import functools

import jax
import jax.numpy as jnp
from jax import lax
from jax.experimental import pallas as pl
from jax.experimental.pallas import tpu as pltpu
from jax.experimental.pallas import tpu_sc as plsc

_NC, _NS, _L = 2, 16, 16
_NW = _NC * _NS

_N = 4096
_B = 32
_LC = 32
_LA = 24
_D = 256
_NITEMS = _N + _B
_NPAD = 4352
_PER_W = _NPAD // _NW
_CH = 4
_NCHUNK = _PER_W // _CH
_NBUF = 2


def _pool_sc_kernel(emb_h, idx_h, coef_h, a_h, out_h, aout_h,
                    idx_v, rows_v, coef_v, acc_v, aidx_v, arows_v,
                    sems, asem):
    w = lax.axis_index("s") * _NC + lax.axis_index("c")
    base = w * _PER_W

    pltpu.sync_copy(idx_h.at[pl.ds(base * _LC, _PER_W * _LC)], idx_v)
    pltpu.sync_copy(coef_h.at[pl.ds(base, _PER_W)], coef_v)
    pltpu.sync_copy(a_h.at[pl.ds(w * _LA, _LA)], aidx_v)
    pltpu.async_copy(emb_h.at[aidx_v], arows_v, asem)

    def _gather(ci, buf):
        return pltpu.make_async_copy(
            emb_h.at[idx_v.at[pl.ds(ci * _CH * _LC, _CH * _LC)]],
            rows_v.at[buf], sems.at[buf])

    def _fire(ci, buf):
        pltpu.async_copy(
            emb_h.at[idx_v.at[pl.ds(ci * _CH * _LC, _CH * _LC)]],
            rows_v.at[buf], sems.at[buf])

    _fire(0, 0)
    _fire(1, 1)

    def outer(it, carry):
        for buf in range(_NBUF):
            ci = it * _NBUF + buf
            _gather(ci, buf).wait()

            def item_body(j, carry2):
                coefj = coef_v[ci * _CH + j]
                r0 = j * _LC
                for ch in range(_D // _L):
                    sl = pl.ds(ch * _L, _L)
                    acc = rows_v[buf, r0, sl] * (1.0 - coefj)
                    for l in range(1, _LC):
                        acc = acc + rows_v[buf, r0 + l, sl]
                    acc_v[buf * _CH + j, sl] = acc
                return carry2

            lax.fori_loop(0, _CH, item_body, 0)

            @pl.when(ci + _NBUF < _NCHUNK)
            def _():
                _fire(ci + _NBUF, buf)

        pltpu.sync_copy(acc_v, out_h.at[pl.ds(base + it * _NBUF * _CH,
                                              _NBUF * _CH)])
        return carry

    lax.fori_loop(0, _NCHUNK // _NBUF, outer, 0)

    pltpu.make_async_copy(emb_h.at[aidx_v], arows_v, asem).wait()
    pltpu.sync_copy(arows_v, aout_h.at[w])


def _gather_top_sc_kernel(emb_h, ctok_h, best_h, cout_h,
                          bidx_v, ctoksel_v, cemb_v, sem):
    w = lax.axis_index("s") * _NC + lax.axis_index("c")
    pltpu.sync_copy(best_h, bidx_v)
    pltpu.async_copy(ctok_h.at[bidx_v], ctoksel_v, sem).wait()
    pltpu.async_copy(emb_h.at[ctoksel_v.at[w, pl.ds(0, _LC)]], cemb_v,
                     sem).wait()
    pltpu.sync_copy(cemb_v, cout_h.at[w])


def _score_tc_kernel(s_ref, best_ref):
    S = s_ref[...]
    cs = S[:_N, :]
    qs = S[_N:_N + _B, :]
    cn = cs * lax.rsqrt(jnp.sum(cs * cs, axis=1, keepdims=True) + 1e-30)
    scores = lax.dot_general(qs, cn, (((1,), (1,)), ((), ())),
                             preferred_element_type=jnp.float32)
    m = jnp.max(scores, axis=1, keepdims=True)
    ii = lax.broadcasted_iota(jnp.int32, scores.shape, 1)
    cand = jnp.where(scores >= m, ii, jnp.int32(2 ** 30))
    best_ref[...] = jnp.min(cand, axis=1)


def _loss_tc_kernel(alen_ref, a_ref, c_ref, out_ref):
    b = pl.program_id(0)
    A = a_ref[0]
    C = c_ref[0]
    an = A * lax.rsqrt(jnp.sum(A * A, axis=1, keepdims=True))
    cn = C * lax.rsqrt(jnp.sum(C * C, axis=1, keepdims=True))
    cos = lax.dot_general(an, cn, (((1,), (1,)), ((), ())),
                          preferred_element_type=jnp.float32)
    em = jnp.exp(-0.5 * (cos - 1.0) ** 2 / (0.001 ** 2))
    sm = em / (jnp.sum(em, axis=1, keepdims=True) + 1e-10)
    mm = jnp.sum(em * sm, axis=1, keepdims=True)
    al = alen_ref[b].astype(jnp.float32)
    mask = (lax.broadcasted_iota(jnp.int32, (_LA, 1), 0)
            < alen_ref[b]).astype(jnp.float32)
    tot = jnp.sum(mm * mask)
    loss_b = 1.0 - tot / al

    @pl.when(b == 0)
    def _():
        out_ref[0, 0] = 0.0

    out_ref[0, 0] += loss_b / _B


def _sc_mesh():
    return plsc.VectorSubcoreMesh(core_axis_name="c", subcore_axis_name="s",
                                  num_cores=_NC, num_subcores=_NS)


def _pool_call(*args):
    return pl.kernel(
        _pool_sc_kernel,
        out_type=(jax.ShapeDtypeStruct((_NPAD, _D), jnp.float32),
                  jax.ShapeDtypeStruct((_B, _LA, _D), jnp.float32)),
        mesh=_sc_mesh(),
        scratch_types=[
            pltpu.VMEM((_PER_W * _LC,), jnp.int32),
            pltpu.VMEM((_NBUF, _CH * _LC, _D), jnp.float32),
            pltpu.VMEM((_PER_W, _L), jnp.float32),
            pltpu.VMEM((_NBUF * _CH, _D), jnp.float32),
            pltpu.VMEM((_LA,), jnp.int32),
            pltpu.VMEM((_LA, _D), jnp.float32),
            pltpu.SemaphoreType.DMA((_NBUF,)),
            pltpu.SemaphoreType.DMA,
        ],
    )(*args)


def _gather_top_call(*args):
    return pl.kernel(
        _gather_top_sc_kernel,
        out_type=jax.ShapeDtypeStruct((_B, _LC, _D), jnp.float32),
        mesh=_sc_mesh(),
        scratch_types=[
            pltpu.VMEM((_B,), jnp.int32),
            pltpu.VMEM((_B, 128), jnp.int32),
            pltpu.VMEM((_LC, _D), jnp.float32),
            pltpu.SemaphoreType.DMA,
        ],
    )(*args)


def kernel(emb, q, c, a, qlen, clen, alen):
    emb = emb.astype(jnp.float32)
    ctok = c[:, :, 0].astype(jnp.int32)
    qtok = q[:, :, 0].astype(jnp.int32)
    clen = clen.astype(jnp.int32)
    qlen = qlen.astype(jnp.int32)
    alen = alen.astype(jnp.int32)
    a32 = a.astype(jnp.int32)

    pos = jnp.arange(_LC, dtype=jnp.int32)[None, :]
    cidx = jnp.where(pos < clen[:, None], ctok, ctok[:, :1])
    qidx = jnp.where(pos < qlen[:, None], qtok, qtok[:, :1])
    ccoef = (_LC - clen).astype(jnp.float32)
    qcoef = (_LC - qlen).astype(jnp.float32)

    idx_all = jnp.concatenate(
        [cidx, qidx, jnp.zeros((_NPAD - _NITEMS, _LC), jnp.int32)], axis=0)
    coef_all = jnp.concatenate(
        [ccoef, qcoef, jnp.zeros((_NPAD - _NITEMS,), jnp.float32)], axis=0)
    idx_flat = idx_all.reshape(-1)
    coef_b = coef_all[:, None] + jnp.zeros((_NPAD, _L), jnp.float32)
    a_flat = a32.reshape(-1)

    ssum, a_emb = _pool_call(emb, idx_flat, coef_b, a_flat)

    best = pl.pallas_call(
        _score_tc_kernel,
        out_shape=jax.ShapeDtypeStruct((_B,), jnp.int32),
    )(ssum)

    ctok_pad = jnp.pad(ctok, ((0, 0), (0, 128 - _LC)))
    c_emb = _gather_top_call(emb, ctok_pad, best)

    loss = pl.pallas_call(
        _loss_tc_kernel,
        grid=(_B,),
        in_specs=[
            pl.BlockSpec(memory_space=pltpu.SMEM),
            pl.BlockSpec((1, _LA, _D), lambda b: (b, 0, 0)),
            pl.BlockSpec((1, _LC, _D), lambda b: (b, 0, 0)),
        ],
        out_specs=pl.BlockSpec(memory_space=pltpu.SMEM),
        out_shape=jax.ShapeDtypeStruct((1, 1), jnp.float32),
    )(alen, a_emb, c_emb)

    return loss[0, 0]


# speedup vs baseline: 2.7014x; 1.0064x over previous
import functools

import jax
import jax.numpy as jnp
from jax import lax
from jax.experimental import pallas as pl
from jax.experimental.pallas import tpu as pltpu
from jax.experimental.pallas import tpu_sc as plsc

_NC, _NS, _L = 2, 16, 16
_NW = _NC * _NS

_N = 4096
_B = 32
_LC = 32
_LA = 24
_D = 256
_NITEMS = _N + _B
_NPAD = 4352
_PER_W = _NPAD // _NW
_CH = 4
_NCHUNK = _PER_W // _CH
_NBUF = 2


def _pool_sc_kernel(emb_h, idx_h, coef_h, a_h, out_h, aout_h,
                    idx_v, rows_v, coef_v, acc_v, aidx_v, arows_v,
                    sems, asem):
    w = lax.axis_index("s") * _NC + lax.axis_index("c")
    base = w * _PER_W

    pltpu.sync_copy(idx_h.at[pl.ds(base * _LC, _PER_W * _LC)], idx_v)
    pltpu.sync_copy(coef_h.at[pl.ds(base, _PER_W)], coef_v)
    pltpu.sync_copy(a_h.at[pl.ds(w * _LA, _LA)], aidx_v)
    pltpu.async_copy(emb_h.at[aidx_v], arows_v, asem)

    def _gather(ci, buf):
        return pltpu.make_async_copy(
            emb_h.at[idx_v.at[pl.ds(ci * _CH * _LC, _CH * _LC)]],
            rows_v.at[buf], sems.at[buf])

    def _fire(ci, buf):
        pltpu.async_copy(
            emb_h.at[idx_v.at[pl.ds(ci * _CH * _LC, _CH * _LC)]],
            rows_v.at[buf], sems.at[buf])

    _fire(0, 0)
    _fire(1, 1)

    def outer(it, carry):
        for buf in range(_NBUF):
            ci = it * _NBUF + buf
            _gather(ci, buf).wait()

            def item_body(j, carry2):
                coefj = coef_v[ci * _CH + j]
                r0 = j * _LC
                for ch in range(_D // _L):
                    sl = pl.ds(ch * _L, _L)
                    accs = [rows_v[buf, r0 + k, sl] for k in range(1, 8)]
                    accs.insert(0, rows_v[buf, r0, sl] * (1.0 - coefj))
                    for lb in range(8, _LC, 8):
                        for k in range(8):
                            accs[k] = accs[k] + rows_v[buf, r0 + lb + k, sl]
                    a0 = (accs[0] + accs[1]) + (accs[2] + accs[3])
                    a1 = (accs[4] + accs[5]) + (accs[6] + accs[7])
                    acc_v[buf * _CH + j, sl] = a0 + a1
                return carry2

            lax.fori_loop(0, _CH, item_body, 0)

            @pl.when(ci + _NBUF < _NCHUNK)
            def _():
                _fire(ci + _NBUF, buf)

        pltpu.sync_copy(acc_v, out_h.at[pl.ds(base + it * _NBUF * _CH,
                                              _NBUF * _CH)])
        return carry

    lax.fori_loop(0, _NCHUNK // _NBUF, outer, 0)

    pltpu.make_async_copy(emb_h.at[aidx_v], arows_v, asem).wait()
    pltpu.sync_copy(arows_v, aout_h.at[w])


def _gather_top_sc_kernel(emb_h, ctok_h, best_h, cout_h,
                          bidx_v, ctoksel_v, cemb_v, sem):
    w = lax.axis_index("s") * _NC + lax.axis_index("c")
    pltpu.sync_copy(best_h, bidx_v)
    pltpu.async_copy(ctok_h.at[bidx_v], ctoksel_v, sem).wait()
    pltpu.async_copy(emb_h.at[ctoksel_v.at[w, pl.ds(0, _LC)]], cemb_v,
                     sem).wait()
    pltpu.sync_copy(cemb_v, cout_h.at[w])


def _score_tc_kernel(s_ref, best_ref):
    S = s_ref[...]
    cs = S[:_N, :]
    qs = S[_N:_N + _B, :]
    cn = cs * lax.rsqrt(jnp.sum(cs * cs, axis=1, keepdims=True) + 1e-30)
    scores = lax.dot_general(qs, cn, (((1,), (1,)), ((), ())),
                             preferred_element_type=jnp.float32)
    m = jnp.max(scores, axis=1, keepdims=True)
    ii = lax.broadcasted_iota(jnp.int32, scores.shape, 1)
    cand = jnp.where(scores >= m, ii, jnp.int32(2 ** 30))
    best_ref[...] = jnp.min(cand, axis=1)


def _loss_tc_kernel(alen_ref, a_ref, c_ref, out_ref):
    b = pl.program_id(0)
    A = a_ref[0]
    C = c_ref[0]
    an = A * lax.rsqrt(jnp.sum(A * A, axis=1, keepdims=True))
    cn = C * lax.rsqrt(jnp.sum(C * C, axis=1, keepdims=True))
    cos = lax.dot_general(an, cn, (((1,), (1,)), ((), ())),
                          preferred_element_type=jnp.float32)
    em = jnp.exp(-0.5 * (cos - 1.0) ** 2 / (0.001 ** 2))
    sm = em / (jnp.sum(em, axis=1, keepdims=True) + 1e-10)
    mm = jnp.sum(em * sm, axis=1, keepdims=True)
    al = alen_ref[b].astype(jnp.float32)
    mask = (lax.broadcasted_iota(jnp.int32, (_LA, 1), 0)
            < alen_ref[b]).astype(jnp.float32)
    tot = jnp.sum(mm * mask)
    loss_b = 1.0 - tot / al

    @pl.when(b == 0)
    def _():
        out_ref[0, 0] = 0.0

    out_ref[0, 0] += loss_b / _B


def _sc_mesh():
    return plsc.VectorSubcoreMesh(core_axis_name="c", subcore_axis_name="s",
                                  num_cores=_NC, num_subcores=_NS)


def _pool_call(*args):
    return pl.kernel(
        _pool_sc_kernel,
        out_type=(jax.ShapeDtypeStruct((_NPAD, _D), jnp.float32),
                  jax.ShapeDtypeStruct((_B, _LA, _D), jnp.float32)),
        mesh=_sc_mesh(),
        scratch_types=[
            pltpu.VMEM((_PER_W * _LC,), jnp.int32),
            pltpu.VMEM((_NBUF, _CH * _LC, _D), jnp.float32),
            pltpu.VMEM((_PER_W, _L), jnp.float32),
            pltpu.VMEM((_NBUF * _CH, _D), jnp.float32),
            pltpu.VMEM((_LA,), jnp.int32),
            pltpu.VMEM((_LA, _D), jnp.float32),
            pltpu.SemaphoreType.DMA((_NBUF,)),
            pltpu.SemaphoreType.DMA,
        ],
    )(*args)


def _gather_top_call(*args):
    return pl.kernel(
        _gather_top_sc_kernel,
        out_type=jax.ShapeDtypeStruct((_B, _LC, _D), jnp.float32),
        mesh=_sc_mesh(),
        scratch_types=[
            pltpu.VMEM((_B,), jnp.int32),
            pltpu.VMEM((_B, 128), jnp.int32),
            pltpu.VMEM((_LC, _D), jnp.float32),
            pltpu.SemaphoreType.DMA,
        ],
    )(*args)


def kernel(emb, q, c, a, qlen, clen, alen):
    emb = emb.astype(jnp.float32)
    ctok = c[:, :, 0].astype(jnp.int32)
    qtok = q[:, :, 0].astype(jnp.int32)
    clen = clen.astype(jnp.int32)
    qlen = qlen.astype(jnp.int32)
    alen = alen.astype(jnp.int32)
    a32 = a.astype(jnp.int32)

    pos = jnp.arange(_LC, dtype=jnp.int32)[None, :]
    cidx = jnp.where(pos < clen[:, None], ctok, ctok[:, :1])
    qidx = jnp.where(pos < qlen[:, None], qtok, qtok[:, :1])
    ccoef = (_LC - clen).astype(jnp.float32)
    qcoef = (_LC - qlen).astype(jnp.float32)

    idx_all = jnp.concatenate(
        [cidx, qidx, jnp.zeros((_NPAD - _NITEMS, _LC), jnp.int32)], axis=0)
    coef_all = jnp.concatenate(
        [ccoef, qcoef, jnp.zeros((_NPAD - _NITEMS,), jnp.float32)], axis=0)
    idx_flat = idx_all.reshape(-1)
    coef_b = coef_all[:, None] + jnp.zeros((_NPAD, _L), jnp.float32)
    a_flat = a32.reshape(-1)

    ssum, a_emb = _pool_call(emb, idx_flat, coef_b, a_flat)

    best = pl.pallas_call(
        _score_tc_kernel,
        out_shape=jax.ShapeDtypeStruct((_B,), jnp.int32),
    )(ssum)

    ctok_pad = jnp.pad(ctok, ((0, 0), (0, 128 - _LC)))
    c_emb = _gather_top_call(emb, ctok_pad, best)

    loss = pl.pallas_call(
        _loss_tc_kernel,
        grid=(_B,),
        in_specs=[
            pl.BlockSpec(memory_space=pltpu.SMEM),
            pl.BlockSpec((1, _LA, _D), lambda b: (b, 0, 0)),
            pl.BlockSpec((1, _LC, _D), lambda b: (b, 0, 0)),
        ],
        out_specs=pl.BlockSpec(memory_space=pltpu.SMEM),
        out_shape=jax.ShapeDtypeStruct((1, 1), jnp.float32),
    )(alen, a_emb, c_emb)

    return loss[0, 0]


# speedup vs baseline: 2.7389x; 1.0139x over previous
import functools

import jax
import jax.numpy as jnp
from jax import lax
from jax.experimental import pallas as pl
from jax.experimental.pallas import tpu as pltpu
from jax.experimental.pallas import tpu_sc as plsc

_NC, _NS, _L = 2, 16, 16
_NW = _NC * _NS

_N = 4096
_B = 32
_LC = 32
_LA = 24
_D = 256
_NITEMS = _N + _B
_NPAD = 4352
_PER_W = _NPAD // _NW
_CH = 2
_NCHUNK = _PER_W // _CH
_NBUF = 4


def _pool_sc_kernel(emb_h, idx_h, coef_h, a_h, out_h, aout_h,
                    idx_v, rows_v, coef_v, acc_v, aidx_v, arows_v,
                    sems, asem):
    w = lax.axis_index("s") * _NC + lax.axis_index("c")
    base = w * _PER_W

    pltpu.sync_copy(idx_h.at[pl.ds(base * _LC, _PER_W * _LC)], idx_v)
    pltpu.sync_copy(coef_h.at[pl.ds(base, _PER_W)], coef_v)
    pltpu.sync_copy(a_h.at[pl.ds(w * _LA, _LA)], aidx_v)
    pltpu.async_copy(emb_h.at[aidx_v], arows_v, asem)

    def _gather(ci, buf):
        return pltpu.make_async_copy(
            emb_h.at[idx_v.at[pl.ds(ci * _CH * _LC, _CH * _LC)]],
            rows_v.at[buf], sems.at[buf])

    def _fire(ci, buf):
        pltpu.async_copy(
            emb_h.at[idx_v.at[pl.ds(ci * _CH * _LC, _CH * _LC)]],
            rows_v.at[buf], sems.at[buf])

    for b0 in range(_NBUF):
        _fire(b0, b0)

    def outer(it, carry):
        for buf in range(_NBUF):
            ci = it * _NBUF + buf
            _gather(ci, buf).wait()

            def item_body(j, carry2):
                coefj = coef_v[ci * _CH + j]
                r0 = j * _LC
                for ch in range(_D // _L):
                    sl = pl.ds(ch * _L, _L)
                    accs = [rows_v[buf, r0 + k, sl] for k in range(1, 8)]
                    accs.insert(0, rows_v[buf, r0, sl] * (1.0 - coefj))
                    for lb in range(8, _LC, 8):
                        for k in range(8):
                            accs[k] = accs[k] + rows_v[buf, r0 + lb + k, sl]
                    a0 = (accs[0] + accs[1]) + (accs[2] + accs[3])
                    a1 = (accs[4] + accs[5]) + (accs[6] + accs[7])
                    acc_v[buf * _CH + j, sl] = a0 + a1
                return carry2

            lax.fori_loop(0, _CH, item_body, 0)

            @pl.when(ci + _NBUF < _NCHUNK)
            def _():
                _fire(ci + _NBUF, buf)

        pltpu.sync_copy(acc_v, out_h.at[pl.ds(base + it * _NBUF * _CH,
                                              _NBUF * _CH)])
        return carry

    lax.fori_loop(0, _NCHUNK // _NBUF, outer, 0)

    pltpu.make_async_copy(emb_h.at[aidx_v], arows_v, asem).wait()
    pltpu.sync_copy(arows_v, aout_h.at[w])


def _gather_top_sc_kernel(emb_h, ctok_h, best_h, cout_h,
                          bidx_v, ctoksel_v, cemb_v, sem):
    w = lax.axis_index("s") * _NC + lax.axis_index("c")
    pltpu.sync_copy(best_h, bidx_v)
    pltpu.async_copy(ctok_h.at[bidx_v], ctoksel_v, sem).wait()
    pltpu.async_copy(emb_h.at[ctoksel_v.at[w, pl.ds(0, _LC)]], cemb_v,
                     sem).wait()
    pltpu.sync_copy(cemb_v, cout_h.at[w])


def _score_tc_kernel(s_ref, best_ref):
    S = s_ref[...]
    cs = S[:_N, :]
    qs = S[_N:_N + _B, :]
    cn = cs * lax.rsqrt(jnp.sum(cs * cs, axis=1, keepdims=True) + 1e-30)
    scores = lax.dot_general(qs, cn, (((1,), (1,)), ((), ())),
                             preferred_element_type=jnp.float32)
    m = jnp.max(scores, axis=1, keepdims=True)
    ii = lax.broadcasted_iota(jnp.int32, scores.shape, 1)
    cand = jnp.where(scores >= m, ii, jnp.int32(2 ** 30))
    best_ref[...] = jnp.min(cand, axis=1)


def _loss_tc_kernel(alen_ref, a_ref, c_ref, out_ref):
    b = pl.program_id(0)
    A = a_ref[0]
    C = c_ref[0]
    an = A * lax.rsqrt(jnp.sum(A * A, axis=1, keepdims=True))
    cn = C * lax.rsqrt(jnp.sum(C * C, axis=1, keepdims=True))
    cos = lax.dot_general(an, cn, (((1,), (1,)), ((), ())),
                          preferred_element_type=jnp.float32)
    em = jnp.exp(-0.5 * (cos - 1.0) ** 2 / (0.001 ** 2))
    sm = em / (jnp.sum(em, axis=1, keepdims=True) + 1e-10)
    mm = jnp.sum(em * sm, axis=1, keepdims=True)
    al = alen_ref[b].astype(jnp.float32)
    mask = (lax.broadcasted_iota(jnp.int32, (_LA, 1), 0)
            < alen_ref[b]).astype(jnp.float32)
    tot = jnp.sum(mm * mask)
    loss_b = 1.0 - tot / al

    @pl.when(b == 0)
    def _():
        out_ref[0, 0] = 0.0

    out_ref[0, 0] += loss_b / _B


def _sc_mesh():
    return plsc.VectorSubcoreMesh(core_axis_name="c", subcore_axis_name="s",
                                  num_cores=_NC, num_subcores=_NS)


def _pool_call(*args):
    return pl.kernel(
        _pool_sc_kernel,
        out_type=(jax.ShapeDtypeStruct((_NPAD, _D), jnp.float32),
                  jax.ShapeDtypeStruct((_B, _LA, _D), jnp.float32)),
        mesh=_sc_mesh(),
        scratch_types=[
            pltpu.VMEM((_PER_W * _LC,), jnp.int32),
            pltpu.VMEM((_NBUF, _CH * _LC, _D), jnp.float32),
            pltpu.VMEM((_PER_W, _L), jnp.float32),
            pltpu.VMEM((_NBUF * _CH, _D), jnp.float32),
            pltpu.VMEM((_LA,), jnp.int32),
            pltpu.VMEM((_LA, _D), jnp.float32),
            pltpu.SemaphoreType.DMA((_NBUF,)),
            pltpu.SemaphoreType.DMA,
        ],
    )(*args)


def _gather_top_call(*args):
    return pl.kernel(
        _gather_top_sc_kernel,
        out_type=jax.ShapeDtypeStruct((_B, _LC, _D), jnp.float32),
        mesh=_sc_mesh(),
        scratch_types=[
            pltpu.VMEM((_B,), jnp.int32),
            pltpu.VMEM((_B, 128), jnp.int32),
            pltpu.VMEM((_LC, _D), jnp.float32),
            pltpu.SemaphoreType.DMA,
        ],
    )(*args)


def kernel(emb, q, c, a, qlen, clen, alen):
    emb = emb.astype(jnp.float32)
    ctok = c[:, :, 0].astype(jnp.int32)
    qtok = q[:, :, 0].astype(jnp.int32)
    clen = clen.astype(jnp.int32)
    qlen = qlen.astype(jnp.int32)
    alen = alen.astype(jnp.int32)
    a32 = a.astype(jnp.int32)

    pos = jnp.arange(_LC, dtype=jnp.int32)[None, :]
    cidx = jnp.where(pos < clen[:, None], ctok, ctok[:, :1])
    qidx = jnp.where(pos < qlen[:, None], qtok, qtok[:, :1])
    ccoef = (_LC - clen).astype(jnp.float32)
    qcoef = (_LC - qlen).astype(jnp.float32)

    idx_all = jnp.concatenate(
        [cidx, qidx, jnp.zeros((_NPAD - _NITEMS, _LC), jnp.int32)], axis=0)
    coef_all = jnp.concatenate(
        [ccoef, qcoef, jnp.zeros((_NPAD - _NITEMS,), jnp.float32)], axis=0)
    idx_flat = idx_all.reshape(-1)
    coef_b = coef_all[:, None] + jnp.zeros((_NPAD, _L), jnp.float32)
    a_flat = a32.reshape(-1)

    ssum, a_emb = _pool_call(emb, idx_flat, coef_b, a_flat)

    best = pl.pallas_call(
        _score_tc_kernel,
        out_shape=jax.ShapeDtypeStruct((_B,), jnp.int32),
    )(ssum)

    ctok_pad = jnp.pad(ctok, ((0, 0), (0, 128 - _LC)))
    c_emb = _gather_top_call(emb, ctok_pad, best)

    loss = pl.pallas_call(
        _loss_tc_kernel,
        grid=(_B,),
        in_specs=[
            pl.BlockSpec(memory_space=pltpu.SMEM),
            pl.BlockSpec((1, _LA, _D), lambda b: (b, 0, 0)),
            pl.BlockSpec((1, _LC, _D), lambda b: (b, 0, 0)),
        ],
        out_specs=pl.BlockSpec(memory_space=pltpu.SMEM),
        out_shape=jax.ShapeDtypeStruct((1, 1), jnp.float32),
    )(alen, a_emb, c_emb)

    return loss[0, 0]
